# baseline (speedup 1.0000x reference)
import functools

import jax
import jax.numpy as jnp
from jax import lax
from jax.experimental import pallas as pl
from jax.experimental.pallas import tpu as pltpu
from jax.experimental.pallas import tpu_sc as plsc

NC = 2
NS = 16
NW = NC * NS

CHUNK = 128
EMBED = 64
LANES = 16

PACK_W = 4096
PACK_NBLK = 123
HALF = PACK_W * (PACK_NBLK - 1)

NBUF = 4
N_S = 50
N_B = 4096


def _pack_body(ta_ref, tb_ref, out_ref):
    out_ref[:, :EMBED] = ta_ref[...].T
    out_ref[:, EMBED:] = tb_ref[...].T


def _pack_table(tt):
    return pl.pallas_call(
        _pack_body,
        grid=(PACK_NBLK,),
        in_specs=[
            pl.BlockSpec((EMBED, PACK_W), lambda j: (0, j)),
            pl.BlockSpec((EMBED, PACK_W), lambda j: (0, j + PACK_NBLK - 1)),
        ],
        out_specs=pl.BlockSpec((PACK_W, 2 * EMBED), lambda j: (j, 0)),
        out_shape=jax.ShapeDtypeStruct((PACK_W * PACK_NBLK, 2 * EMBED),
                                       jnp.float32),
        compiler_params=pltpu.CompilerParams(
            dimension_semantics=("parallel",)),
    )(tt, tt)


def _gather_body(ids_hbm, tab_hbm, out_hbm, idxv, qv, hv,
                 a_bufs, b_bufs, gsems, wsems):
    cid = lax.axis_index("c")
    sid = lax.axis_index("s")
    wid = sid * NC + cid

    iota = lax.iota(jnp.int32, LANES)
    row16 = [g * LANES + iota for g in range(CHUNK // LANES)]

    pltpu.sync_copy(ids_hbm.at[:, pl.ds(wid * CHUNK, CHUNK)], idxv)

    def stage(i, _):
        r16 = jnp.full((LANES,), i // 8, jnp.int32)
        c16 = (i % 8) * LANES + iota
        v = plsc.load_gather(idxv, [r16, c16])
        q = jnp.where(v < HALF, v, v - HALF)
        h = jnp.where(v < HALF, 0, EMBED).astype(jnp.int32)
        qv[pl.ds(i * LANES, LANES)] = q
        hv[pl.ds(i * LANES, LANES)] = h
        return _

    lax.fori_loop(0, (N_S * CHUNK) // LANES, stage, None)

    def gather(j, b):
        return pltpu.make_async_copy(
            tab_hbm.at[qv.at[pl.ds(j * CHUNK, CHUNK)]], a_bufs[b], gsems[b])

    def write(j, p):
        return pltpu.make_async_copy(
            b_bufs[p], out_hbm.at[j, :, pl.ds(wid * CHUNK, CHUNK)], wsems[p])

    def extract(j, a_ref, b_ref):
        h16s = [hv[pl.ds(j * CHUNK + g * LANES, LANES)]
                for g in range(CHUNK // LANES)]

        def col(c, _):
            vals = [plsc.load_gather(a_ref, [row16[g], h16s[g] + c])
                    for g in range(CHUNK // LANES)]
            for g in range(CHUNK // LANES):
                b_ref[c, pl.ds(g * LANES, LANES)] = vals[g]
            return _

        lax.fori_loop(0, EMBED, col, None)

    def step(j, b):
        gather(j, b).wait()

        @pl.when(j >= 2)
        def _():
            write(j - 2, b % 2).wait()

        extract(j, a_bufs[b], b_bufs[b % 2])
        write(j, b % 2).start()

        @pl.when(j + NBUF < N_S)
        def _():
            gather(j + NBUF, b).start()

    for b in range(NBUF):
        gather(b, b).start()

    def outer(o, _):
        for b in range(NBUF):
            step(o * NBUF + b, b)
        return _

    lax.fori_loop(0, N_S // NBUF, outer, None)
    for j in range(N_S - N_S % NBUF, N_S):
        step(j, j % NBUF)

    write(N_S - 2, (N_S - 2) % 2).wait()
    write(N_S - 1, (N_S - 1) % 2).wait()


def _make_gather():
    def body(ids_hbm, tab_hbm, out_hbm, *scratch):
        idxv, qv, hv = scratch[0], scratch[1], scratch[2]
        a_bufs = scratch[3:3 + NBUF]
        b_bufs = scratch[3 + NBUF:5 + NBUF]
        gsems = scratch[5 + NBUF:5 + 2 * NBUF]
        wsems = scratch[5 + 2 * NBUF:]
        _gather_body(ids_hbm, tab_hbm, out_hbm, idxv, qv, hv,
                     a_bufs, b_bufs, gsems, wsems)

    return pl.kernel(
        body,
        out_type=jax.ShapeDtypeStruct((N_S, EMBED, N_B), jnp.float32),
        mesh=plsc.VectorSubcoreMesh(core_axis_name="c", subcore_axis_name="s"),
        scratch_types=[
            pltpu.VMEM((N_S, CHUNK), jnp.int32),
            pltpu.VMEM((N_S * CHUNK,), jnp.int32),
            pltpu.VMEM((N_S * CHUNK,), jnp.int32),
        ]
        + [pltpu.VMEM((CHUNK, 2 * EMBED), jnp.float32)] * NBUF
        + [pltpu.VMEM((EMBED, CHUNK), jnp.float32)] * 2
        + [pltpu.SemaphoreType.DMA] * (NBUF + 2),
        compiler_params=pltpu.CompilerParams(needs_layout_passes=False),
    )


@jax.jit
def kernel(input_ids, table):
    ids_t = input_ids.astype(jnp.int32).T
    packed_tab = _pack_table(table.T)
    out3 = _make_gather()(ids_t, packed_tab)
    return jnp.transpose(out3, (2, 0, 1))


# speedup vs baseline: 1.0651x; 1.0651x over previous
import functools

import jax
import jax.numpy as jnp
from jax import lax
from jax.experimental import pallas as pl
from jax.experimental.pallas import tpu as pltpu
from jax.experimental.pallas import tpu_sc as plsc

NC = 2
NS = 16
NW = NC * NS

CHUNK = 128
EMBED = 64
LANES = 16

PACK_W = 8192
PACK_NBLK = 62
HALF = PACK_W * (PACK_NBLK - 1)

NBUF = 4
N_S = 50
N_B = 4096


def _pack_body(ta_ref, tb_ref, out_ref):
    out_ref[:, :EMBED] = ta_ref[...].T
    out_ref[:, EMBED:] = tb_ref[...].T


def _pack_table(tt):
    return pl.pallas_call(
        _pack_body,
        grid=(PACK_NBLK,),
        in_specs=[
            pl.BlockSpec((EMBED, PACK_W), lambda j: (0, j)),
            pl.BlockSpec((EMBED, PACK_W), lambda j: (0, j + PACK_NBLK - 1)),
        ],
        out_specs=pl.BlockSpec((PACK_W, 2 * EMBED), lambda j: (j, 0)),
        out_shape=jax.ShapeDtypeStruct((PACK_W * PACK_NBLK, 2 * EMBED),
                                       jnp.float32),
        compiler_params=pltpu.CompilerParams(
            dimension_semantics=("parallel",)),
    )(tt, tt)


def _gather_body(ids_hbm, tab_hbm, out_hbm, idxv, qv, hv,
                 a_bufs, b_bufs, gsems, wsems):
    cid = lax.axis_index("c")
    sid = lax.axis_index("s")
    wid = sid * NC + cid

    iota = lax.iota(jnp.int32, LANES)
    row16 = [g * LANES + iota for g in range(CHUNK // LANES)]

    pltpu.sync_copy(ids_hbm.at[:, pl.ds(wid * CHUNK, CHUNK)], idxv)

    def stage(i, _):
        r16 = jnp.full((LANES,), i // 8, jnp.int32)
        c16 = (i % 8) * LANES + iota
        v = plsc.load_gather(idxv, [r16, c16])
        q = jnp.where(v < HALF, v, v - HALF)
        h = jnp.where(v < HALF, 0, EMBED).astype(jnp.int32)
        qv[pl.ds(i * LANES, LANES)] = q
        hv[pl.ds(i * LANES, LANES)] = h
        return _

    lax.fori_loop(0, (N_S * CHUNK) // LANES, stage, None)

    def gather(j, b):
        return pltpu.make_async_copy(
            tab_hbm.at[qv.at[pl.ds(j * CHUNK, CHUNK)]], a_bufs[b], gsems[b])

    def write(j, p):
        return pltpu.make_async_copy(
            b_bufs[p], out_hbm.at[j, :, pl.ds(wid * CHUNK, CHUNK)], wsems[p])

    def extract(j, a_ref, b_ref):
        h16s = [hv[pl.ds(j * CHUNK + g * LANES, LANES)]
                for g in range(CHUNK // LANES)]

        def col(c, _):
            vals = [plsc.load_gather(a_ref, [row16[g], h16s[g] + c])
                    for g in range(CHUNK // LANES)]
            for g in range(CHUNK // LANES):
                b_ref[c, pl.ds(g * LANES, LANES)] = vals[g]
            return _

        lax.fori_loop(0, EMBED, col, None)

    def step(j, b):
        gather(j, b).wait()

        @pl.when(j >= 2)
        def _():
            write(j - 2, b % 2).wait()

        extract(j, a_bufs[b], b_bufs[b % 2])
        write(j, b % 2).start()

        @pl.when(j + NBUF < N_S)
        def _():
            gather(j + NBUF, b).start()

    for b in range(NBUF):
        gather(b, b).start()

    def outer(o, _):
        for b in range(NBUF):
            step(o * NBUF + b, b)
        return _

    lax.fori_loop(0, N_S // NBUF, outer, None)
    for j in range(N_S - N_S % NBUF, N_S):
        step(j, j % NBUF)

    write(N_S - 2, (N_S - 2) % 2).wait()
    write(N_S - 1, (N_S - 1) % 2).wait()


def _make_gather():
    def body(ids_hbm, tab_hbm, out_hbm, *scratch):
        idxv, qv, hv = scratch[0], scratch[1], scratch[2]
        a_bufs = scratch[3:3 + NBUF]
        b_bufs = scratch[3 + NBUF:5 + NBUF]
        gsems = scratch[5 + NBUF:5 + 2 * NBUF]
        wsems = scratch[5 + 2 * NBUF:]
        _gather_body(ids_hbm, tab_hbm, out_hbm, idxv, qv, hv,
                     a_bufs, b_bufs, gsems, wsems)

    return pl.kernel(
        body,
        out_type=jax.ShapeDtypeStruct((N_S, EMBED, N_B), jnp.float32),
        mesh=plsc.VectorSubcoreMesh(core_axis_name="c", subcore_axis_name="s"),
        scratch_types=[
            pltpu.VMEM((N_S, CHUNK), jnp.int32),
            pltpu.VMEM((N_S * CHUNK,), jnp.int32),
            pltpu.VMEM((N_S * CHUNK,), jnp.int32),
        ]
        + [pltpu.VMEM((CHUNK, 2 * EMBED), jnp.float32)] * NBUF
        + [pltpu.VMEM((EMBED, CHUNK), jnp.float32)] * 2
        + [pltpu.SemaphoreType.DMA] * (NBUF + 2),
        compiler_params=pltpu.CompilerParams(needs_layout_passes=False),
    )


@jax.jit
def kernel(input_ids, table):
    ids_t = input_ids.astype(jnp.int32).T
    packed_tab = _pack_table(table.T)
    out3 = _make_gather()(ids_t, packed_tab)
    return jnp.transpose(out3, (2, 0, 1))
